# baseline (speedup 1.0000x reference)
import functools

import jax
import jax.numpy as jnp
from jax import lax
from jax.experimental import pallas as pl
from jax.experimental.pallas import tpu as pltpu
from jax.experimental.pallas import tpu_sc as plsc

_F = 8
_IW = 128
_CHUNK_IW = 5


@functools.lru_cache(maxsize=None)
def _build(n_rows: int, vocab: int, d: int):
    info = plsc.get_sparse_core_info()
    num_workers = info.num_cores * info.num_subcores
    chunk_rows = _CHUNK_IW * _IW
    assert n_rows % (num_workers * chunk_rows) == 0
    rows_per_worker = n_rows // num_workers
    n_chunks = rows_per_worker // chunk_rows
    assert n_chunks % 2 == 0

    mesh = plsc.VectorSubcoreMesh(core_axis_name="c", subcore_axis_name="s")

    @functools.partial(
        pl.kernel,
        mesh=mesh,
        compiler_params=pltpu.CompilerParams(use_tc_tiling_on_sc=False),
        out_type=jax.ShapeDtypeStruct((n_rows, d), jnp.float32),
        scratch_types=[
            pltpu.VMEM((2, chunk_rows), jnp.int32),
            pltpu.VMEM((2, chunk_rows, d), jnp.float32),
            pltpu.SemaphoreType.DMA,
            pltpu.SemaphoreType.DMA,
            pltpu.SemaphoreType.DMA,
            pltpu.SemaphoreType.DMA,
        ],
    )
    def gather_kernel(x_hbm, tab_hbm, out_hbm, idx_v, rows_v, g0, g1, o0, o1):
        sem_g = [g0, g1]
        sem_o = [o0, o1]
        wid = lax.axis_index("s") * info.num_cores + lax.axis_index("c")
        base = wid * rows_per_worker
        off_vec = (jnp.arange(16, dtype=jnp.int32) & (_F - 1)) * vocab

        def row0_of(c):
            return pl.multiple_of(base + c * chunk_rows, chunk_rows)

        def prep(c, b):
            row0 = row0_of(c)
            ib = idx_v.at[b]
            pltpu.sync_copy(x_hbm.at[pl.ds(row0, chunk_rows)], ib)
            for t in range(chunk_rows // 16):
                sl = pl.ds(t * 16, 16)
                idx_v[b, sl] = idx_v[b, sl] + off_vec
            for j in range(_CHUNK_IW):
                pltpu.async_copy(
                    tab_hbm.at[ib.at[pl.ds(j * _IW, _IW)]],
                    rows_v.at[b, pl.ds(j * _IW, _IW)],
                    sem_g[b],
                )

        def wait_gathers(b):
            pltpu.make_async_copy(
                tab_hbm.at[pl.ds(0, chunk_rows)], rows_v.at[b], sem_g[b]
            ).wait()

        def fire_out(c, b):
            pltpu.async_copy(rows_v.at[b], out_hbm.at[pl.ds(row0_of(c), chunk_rows)], sem_o[b])

        def wait_out(b):
            pltpu.make_async_copy(
                rows_v.at[b], out_hbm.at[pl.ds(0, chunk_rows)], sem_o[b]
            ).wait()

        def pair_body(i, carry):
            c0 = 2 * i

            @pl.when(i > 0)
            def _():
                wait_out(1)
            prep(c0 + 1, 1)
            wait_gathers(0)
            fire_out(c0, 0)

            @pl.when(i < n_chunks // 2 - 1)
            def _():
                wait_out(0)
                prep(c0 + 2, 0)
            wait_gathers(1)
            fire_out(c0 + 1, 1)
            return carry

        prep(0, 0)
        lax.fori_loop(0, n_chunks // 2, pair_body, 0)
        wait_out(0)
        wait_out(1)

    return gather_kernel


def kernel(x, tables):
    n_split = 4
    b, l, f = x.shape
    n_tab, vocab, d = tables.shape
    n_rows = b * l * f
    rows_per_split = n_rows // n_split
    x_flat = x.reshape(n_rows)
    tab_flat = tables.reshape(n_tab * vocab, d)
    g = _build(rows_per_split, vocab, d)
    parts = [
        g(x_flat[s * rows_per_split:(s + 1) * rows_per_split], tab_flat)
        for s in range(n_split)
    ]
    out = jnp.concatenate(parts, axis=0)
    return out.reshape(b, l, f * d)


# speedup vs baseline: 2.1456x; 2.1456x over previous
import functools

import jax
import jax.numpy as jnp
from jax import lax
from jax.experimental import pallas as pl
from jax.experimental.pallas import tpu as pltpu
from jax.experimental.pallas import tpu_sc as plsc

_F = 8
_IW = 128
_CHUNK_IW = 5


@functools.lru_cache(maxsize=None)
def _build(n_rows: int, vocab: int, d: int):
    info = plsc.get_sparse_core_info()
    num_workers = info.num_cores * info.num_subcores
    chunk_rows = _CHUNK_IW * _IW
    assert n_rows % (num_workers * chunk_rows) == 0
    rows_per_worker = n_rows // num_workers
    n_chunks = rows_per_worker // chunk_rows
    assert n_chunks % 2 == 0

    mesh = plsc.VectorSubcoreMesh(core_axis_name="c", subcore_axis_name="s")

    @functools.partial(
        pl.kernel,
        mesh=mesh,
        compiler_params=pltpu.CompilerParams(use_tc_tiling_on_sc=False),
        out_type=jax.ShapeDtypeStruct((n_rows, d), jnp.float32),
        scratch_types=[
            pltpu.VMEM((2, chunk_rows), jnp.int32),
            pltpu.VMEM((2, chunk_rows, d), jnp.float32),
            pltpu.SemaphoreType.DMA,
            pltpu.SemaphoreType.DMA,
            pltpu.SemaphoreType.DMA,
            pltpu.SemaphoreType.DMA,
        ],
    )
    def gather_kernel(x_hbm, tab_hbm, out_hbm, idx_v, rows_v, g0, g1, o0, o1):
        sem_g = [g0, g1]
        sem_o = [o0, o1]
        wid = lax.axis_index("s") * info.num_cores + lax.axis_index("c")
        base = wid * rows_per_worker
        off_vec = (jnp.arange(16, dtype=jnp.int32) & (_F - 1)) * vocab

        def row0_of(c):
            return pl.multiple_of(base + c * chunk_rows, chunk_rows)

        def prep(c, b):
            row0 = row0_of(c)
            ib = idx_v.at[b]
            pltpu.sync_copy(x_hbm.at[pl.ds(row0, chunk_rows)], ib)
            for t in range(chunk_rows // 16):
                sl = pl.ds(t * 16, 16)
                idx_v[b, sl] = idx_v[b, sl] + off_vec
            for j in range(_CHUNK_IW):
                pltpu.async_copy(
                    tab_hbm.at[ib.at[pl.ds(j * _IW, _IW)]],
                    rows_v.at[b, pl.ds(j * _IW, _IW)],
                    sem_g[b],
                )

        def wait_gathers(b):
            pltpu.make_async_copy(
                tab_hbm.at[pl.ds(0, chunk_rows)], rows_v.at[b], sem_g[b]
            ).wait()

        def fire_out(c, b):
            pltpu.async_copy(rows_v.at[b], out_hbm.at[pl.ds(row0_of(c), chunk_rows)], sem_o[b])

        def wait_out(b):
            pltpu.make_async_copy(
                rows_v.at[b], out_hbm.at[pl.ds(0, chunk_rows)], sem_o[b]
            ).wait()

        def pair_body(i, carry):
            c0 = 2 * i

            @pl.when(i > 0)
            def _():
                wait_out(1)
            prep(c0 + 1, 1)
            wait_gathers(0)
            fire_out(c0, 0)

            @pl.when(i < n_chunks // 2 - 1)
            def _():
                wait_out(0)
                prep(c0 + 2, 0)
            wait_gathers(1)
            fire_out(c0 + 1, 1)
            return carry

        prep(0, 0)
        lax.fori_loop(0, n_chunks // 2, pair_body, 0)
        wait_out(0)
        wait_out(1)

    return gather_kernel


def kernel(x, tables):
    b, l, f = x.shape
    n_tab, vocab, d = tables.shape
    n_rows = b * l * f
    x_flat = x.reshape(n_rows)
    tab_flat = tables.reshape(n_tab * vocab, d)
    out = _build(n_rows, vocab, d)(x_flat, tab_flat)
    return out.reshape(b, l, f * d)


# speedup vs baseline: 2.1465x; 1.0004x over previous
import functools

import jax
import jax.numpy as jnp
from jax import lax
from jax.experimental import pallas as pl
from jax.experimental.pallas import tpu as pltpu
from jax.experimental.pallas import tpu_sc as plsc

_F = 8
_STREAMS = (128, 128, 128, 128, 128, 128, 32)


@functools.lru_cache(maxsize=None)
def _build(n_rows: int, vocab: int, d: int):
    info = plsc.get_sparse_core_info()
    num_workers = info.num_cores * info.num_subcores
    chunk_rows = sum(_STREAMS)
    assert n_rows % (num_workers * chunk_rows) == 0
    rows_per_worker = n_rows // num_workers
    n_chunks = rows_per_worker // chunk_rows
    assert n_chunks % 2 == 0

    mesh = plsc.VectorSubcoreMesh(core_axis_name="c", subcore_axis_name="s")

    @functools.partial(
        pl.kernel,
        mesh=mesh,
        compiler_params=pltpu.CompilerParams(use_tc_tiling_on_sc=False),
        out_type=jax.ShapeDtypeStruct((n_rows, d), jnp.float32),
        scratch_types=[
            pltpu.VMEM((2, chunk_rows), jnp.int32),
            pltpu.VMEM((2, chunk_rows, d), jnp.float32),
            pltpu.SemaphoreType.DMA,
            pltpu.SemaphoreType.DMA,
            pltpu.SemaphoreType.DMA,
            pltpu.SemaphoreType.DMA,
        ],
    )
    def gather_kernel(x_hbm, tab_hbm, out_hbm, idx_v, rows_v, g0, g1, o0, o1):
        sem_g = [g0, g1]
        sem_o = [o0, o1]
        wid = lax.axis_index("s") * info.num_cores + lax.axis_index("c")
        base = wid * rows_per_worker
        off_vec = (jnp.arange(16, dtype=jnp.int32) & (_F - 1)) * vocab

        def row0_of(c):
            return pl.multiple_of(base + c * chunk_rows, chunk_rows)

        def prep(c, b):
            row0 = row0_of(c)
            ib = idx_v.at[b]
            pltpu.sync_copy(x_hbm.at[pl.ds(row0, chunk_rows)], ib)
            for t in range(chunk_rows // 16):
                sl = pl.ds(t * 16, 16)
                idx_v[b, sl] = idx_v[b, sl] + off_vec
            pos = 0
            for sz in _STREAMS:
                pltpu.async_copy(
                    tab_hbm.at[ib.at[pl.ds(pos, sz)]],
                    rows_v.at[b, pl.ds(pos, sz)],
                    sem_g[b],
                )
                pos += sz

        def wait_gathers(b):
            pltpu.make_async_copy(
                tab_hbm.at[pl.ds(0, chunk_rows)], rows_v.at[b], sem_g[b]
            ).wait()

        def fire_out(c, b):
            pltpu.async_copy(rows_v.at[b], out_hbm.at[pl.ds(row0_of(c), chunk_rows)], sem_o[b])

        def wait_out(b):
            pltpu.make_async_copy(
                rows_v.at[b], out_hbm.at[pl.ds(0, chunk_rows)], sem_o[b]
            ).wait()

        def pair_body(i, carry):
            c0 = 2 * i

            @pl.when(i > 0)
            def _():
                wait_out(1)
            prep(c0 + 1, 1)
            wait_gathers(0)
            fire_out(c0, 0)

            @pl.when(i < n_chunks // 2 - 1)
            def _():
                wait_out(0)
                prep(c0 + 2, 0)
            wait_gathers(1)
            fire_out(c0 + 1, 1)
            return carry

        prep(0, 0)
        lax.fori_loop(0, n_chunks // 2, pair_body, 0)
        wait_out(0)
        wait_out(1)

    return gather_kernel


def kernel(x, tables):
    b, l, f = x.shape
    n_tab, vocab, d = tables.shape
    n_rows = b * l * f
    x_flat = x.reshape(n_rows)
    tab_flat = tables.reshape(n_tab * vocab, d)
    out = _build(n_rows, vocab, d)(x_flat, tab_flat)
    return out.reshape(b, l, f * d)


# speedup vs baseline: 2.1487x; 1.0011x over previous
import functools

import jax
import jax.numpy as jnp
from jax import lax
from jax.experimental import pallas as pl
from jax.experimental.pallas import tpu as pltpu
from jax.experimental.pallas import tpu_sc as plsc

_F = 8
_STREAMS = (128, 128, 128, 128, 128, 128, 32)


@functools.lru_cache(maxsize=None)
def _build(n_rows: int, vocab: int, d: int):
    info = plsc.get_sparse_core_info()
    num_workers = info.num_cores * info.num_subcores
    chunk_rows = sum(_STREAMS)
    assert n_rows % (num_workers * chunk_rows) == 0
    rows_per_worker = n_rows // num_workers
    n_chunks = rows_per_worker // chunk_rows
    assert n_chunks % 2 == 0

    mesh = plsc.VectorSubcoreMesh(core_axis_name="c", subcore_axis_name="s")

    @functools.partial(
        pl.kernel,
        mesh=mesh,
        compiler_params=pltpu.CompilerParams(use_tc_tiling_on_sc=False),
        out_type=jax.ShapeDtypeStruct((n_rows, d), jnp.float32),
        scratch_types=[
            pltpu.VMEM((2, chunk_rows), jnp.int32),
            pltpu.VMEM((2, chunk_rows, d), jnp.float32),
            pltpu.SemaphoreType.DMA,
            pltpu.SemaphoreType.DMA,
            pltpu.SemaphoreType.DMA,
            pltpu.SemaphoreType.DMA,
        ],
    )
    def gather_kernel(x_hbm, tab_hbm, out_hbm, idx_v, rows_v, g0, g1, o0, o1):
        sem_g = [g0, g1]
        sem_o = [o0, o1]
        wid = lax.axis_index("s") * info.num_cores + lax.axis_index("c")
        base = wid * rows_per_worker
        off_vec = (jnp.arange(16, dtype=jnp.int32) & (_F - 1)) * vocab

        def row0_of(c):
            return pl.multiple_of(base + c * chunk_rows, chunk_rows)

        def prep(c, b):
            row0 = row0_of(c)
            ib = idx_v.at[b]
            pltpu.sync_copy(x_hbm.at[pl.ds(row0, chunk_rows)], ib)
            for t in range(chunk_rows // 16):
                sl = pl.ds(t * 16, 16)
                idx_v[b, sl] = idx_v[b, sl] + off_vec
            pos = 0
            for sz in _STREAMS:
                pltpu.async_copy(
                    tab_hbm.at[ib.at[pl.ds(pos, sz)]],
                    rows_v.at[b, pl.ds(pos, sz)],
                    sem_g[b],
                )
                pos += sz

        def wait_gathers(b):
            pltpu.make_async_copy(
                tab_hbm.at[pl.ds(0, chunk_rows)], rows_v.at[b], sem_g[b]
            ).wait()

        def fire_out(c, b):
            pltpu.async_copy(rows_v.at[b], out_hbm.at[pl.ds(row0_of(c), chunk_rows)], sem_o[b])

        def wait_out(b):
            pltpu.make_async_copy(
                rows_v.at[b], out_hbm.at[pl.ds(0, chunk_rows)], sem_o[b]
            ).wait()

        def pair_body(i, carry):
            c0 = 2 * i

            @pl.when(i > 0)
            def _():
                wait_out(1)
            prep(c0 + 1, 1)
            wait_gathers(0)
            fire_out(c0, 0)

            @pl.when(i < n_chunks // 2 - 1)
            def _():
                wait_out(0)
                prep(c0 + 2, 0)
            wait_gathers(1)
            fire_out(c0 + 1, 1)
            return carry

        prep(0, 0)
        lax.fori_loop(0, n_chunks // 2, pair_body, 0)
        wait_out(0)
        wait_out(1)

    return gather_kernel


def kernel(x, tables):
    b, l, f = x.shape
    n_tab, vocab, d = tables.shape
    n_rows = b * l * f
    x_flat = x.reshape(n_rows)
    tab_flat = tables.reshape(n_tab * vocab, d)
    x_flat, tab_flat = lax.optimization_barrier((x_flat, tab_flat))
    out = _build(n_rows, vocab, d)(x_flat, tab_flat)
    return out.reshape(b, l, f * d)
